# baseline (speedup 1.0000x reference)
import jax
import jax.numpy as jnp
from jax import lax
from jax.experimental import pallas as pl
from jax.experimental.pallas import tpu as pltpu
from jax.experimental.pallas import tpu_sc as plsc

N = 320000
D = 128
G = 2048
NC = 2
NS = 16
NW = NC * NS
SEG_PER_W = G // NW
T = 400
LANES = 16
NVR = D // LANES


def _wmr_body(h_hbm, pk_hbm, table_hbm, offs_hbm, out_hbm,
              ht0, ht1, pk0, pk1, acc, dacc, table_v, offs_v,
              table_s,
              sem0, sem1):
    wid = lax.axis_index("s") * NC + lax.axis_index("c")
    g0d = wid * SEG_PER_W * D

    pltpu.sync_copy(table_hbm, table_v)
    pltpu.sync_copy(offs_hbm, offs_v)
    tv = table_v[pl.ds(0, LANES)]
    table_s[0] = tv[0]
    table_s[1] = tv[1]
    table_s[2] = tv[2]
    ov = offs_v[pl.ds(wid, LANES)]
    r0 = ov[0]
    r1 = ov[1]

    zeros = jnp.zeros((LANES,), jnp.float32)
    lane0_f = jnp.where(lax.iota(jnp.int32, LANES) == 0, 1.0, 0.0)

    t0 = r0 // T
    t1 = (r1 + T - 1) // T
    nt = t1 - t0

    bufs = ((ht0, pk0, sem0), (ht1, pk1, sem1))

    def issue(t, buf):
        htb, pkb, sem = buf
        base = t * T
        pltpu.async_copy(h_hbm.at[pl.ds(base * D, T * D)], htb, sem)
        pltpu.async_copy(pk_hbm.at[pl.ds(base, T)], pkb.at[pl.ds(0, T)], sem)

    def drain(buf):
        htb, pkb, sem = buf
        pltpu.make_async_copy(h_hbm.at[pl.ds(0, T * D)], htb, sem).wait()
        pltpu.make_async_copy(pk_hbm.at[pl.ds(0, T)], pkb.at[pl.ds(0, T)],
                              sem).wait()

    @pl.when(nt > 0)
    def _():
        issue(t0, bufs[0])

    @plsc.parallel_loop(0, SEG_PER_W * D // LANES, unroll=8)
    def _(k):
        acc[pl.ds(k * LANES, LANES)] = zeros

    @plsc.parallel_loop(0, SEG_PER_W, unroll=8)
    def _(l):
        dacc[pl.ds(l * LANES, LANES)] = zeros

    def do_row(htb, pk, i):
        p = pk & 3
        ao = (pk >> 2) - g0d
        a = table_s[p]
        plsc.addupdate(dacc.at[pl.ds(ao >> 3, LANES)], a * lane0_f)
        ho = i * D
        for j in range(NVR):
            plsc.addupdate(acc.at[pl.ds(ao + j * LANES, LANES)],
                           a * htb[pl.ds(ho + j * LANES, LANES)])

    def process(tt, buf):
        htb, pkb, _ = buf
        base = (t0 + tt) * T
        i_lo = jnp.maximum(r0 - base, 0)
        i_hi = jnp.minimum(r1 - base, T)
        a_lo = (i_lo + LANES - 1) & ~(LANES - 1)
        a_hi = i_hi & ~(LANES - 1)
        mid_end = jnp.minimum(a_lo, i_hi)
        tail_lo = jnp.maximum(a_hi, mid_end)
        blk_hi = jnp.maximum(a_lo, a_hi) >> 4

        @plsc.parallel_loop(i_lo, mid_end)
        def _(i):
            do_row(htb, pkb[pl.ds(i, LANES)][0], i)

        @plsc.parallel_loop(a_lo >> 4, blk_hi)
        def _(b):
            ib = b * LANES
            pkv = pkb[pl.ds(ib, LANES)]
            e0 = pkv[0]
            e15 = pkv[15]
            same = (e0 >> 9) == (e15 >> 9)

            @pl.when(same)
            def _():
                ao = (e0 >> 2) - g0d
                accs = [zeros] * NVR
                for r in range(LANES):
                    a = table_s[pkv[r] & 3]
                    ho = (ib + r) * D
                    for j in range(NVR):
                        accs[j] = accs[j] + a * htb[pl.ds(ho + j * LANES,
                                                          LANES)]
                for j in range(NVR):
                    plsc.addupdate(acc.at[pl.ds(ao + j * LANES, LANES)],
                                   accs[j])
                a16 = plsc.load_gather(table_v, [pkv & 3])
                plsc.addupdate(dacc.at[pl.ds(ao >> 3, LANES)], a16)

            @pl.when(jnp.logical_not(same))
            def _():
                seg0 = e0 >> 9
                seg15 = e15 >> 9
                aoA = (e0 >> 2) - g0d
                aoB = (e15 >> 2) - g0d
                accA = [zeros] * NVR
                accB = [zeros] * NVR
                bad = jnp.int32(0)
                for r in range(LANES):
                    pk = pkv[r]
                    seg_r = pk >> 9
                    a = table_s[pk & 3]
                    inA = seg_r == seg0
                    inB = seg_r == seg15
                    aA = jnp.where(inA, a, 0.0)
                    aB = jnp.where(inB, a, 0.0)
                    bad = bad | jnp.where(jnp.logical_or(inA, inB), 0, 1)
                    ho = (ib + r) * D
                    for j in range(NVR):
                        hv = htb[pl.ds(ho + j * LANES, LANES)]
                        accA[j] = accA[j] + aA * hv
                        accB[j] = accB[j] + aB * hv
                for j in range(NVR):
                    plsc.addupdate(acc.at[pl.ds(aoA + j * LANES, LANES)],
                                   accA[j])
                    plsc.addupdate(acc.at[pl.ds(aoB + j * LANES, LANES)],
                                   accB[j])
                svv = pkv >> 9
                a16 = plsc.load_gather(table_v, [pkv & 3])
                mA = svv == jnp.full((LANES,), seg0)
                mB = svv == jnp.full((LANES,), seg15)
                plsc.addupdate(dacc.at[pl.ds(aoA >> 3, LANES)],
                               jnp.where(mA, a16, 0.0))
                plsc.addupdate(dacc.at[pl.ds(aoB >> 3, LANES)],
                               jnp.where(mB, a16, 0.0))

                @pl.when(bad != 0)
                def _():
                    def mid_row(r, carry):
                        pk = pkb[pl.ds(ib + r, LANES)][0]
                        seg_r = pk >> 9

                        @pl.when(jnp.logical_and(seg_r != seg0,
                                                 seg_r != seg15))
                        def _():
                            do_row(htb, pk, ib + r)
                        return carry

                    lax.fori_loop(0, LANES, mid_row, 0)

        @plsc.parallel_loop(tail_lo, i_hi)
        def _(i):
            do_row(htb, pkb[pl.ds(i, LANES)][0], i)

    def tile_body(tt, carry):
        for k in (0, 1):
            @pl.when((tt & 1) == k)
            def _():
                drain(bufs[k])

                @pl.when(tt + 1 < nt)
                def _():
                    issue(t0 + tt + 1, bufs[1 - k])

                process(tt, bufs[k])
        return carry

    lax.fori_loop(0, nt, tile_body, 0)

    @plsc.parallel_loop(0, SEG_PER_W, unroll=2)
    def _(l):
        d = jnp.sum(dacc[pl.ds(l * LANES, LANES)])
        r16 = 1.0 / jnp.maximum(jnp.full((LANES,), d), 1e-12)
        for j in range(NVR):
            o = l * D + j * LANES
            acc[pl.ds(o, LANES)] = acc[pl.ds(o, LANES)] * r16

    pltpu.sync_copy(acc, out_hbm.at[pl.ds(wid * SEG_PER_W * D, SEG_PER_W * D)])


_wmr = pl.kernel(
    _wmr_body,
    mesh=plsc.VectorSubcoreMesh(core_axis_name="c", subcore_axis_name="s"),
    out_type=jax.ShapeDtypeStruct((G * D,), jnp.float32),
    compiler_params=pltpu.CompilerParams(needs_layout_passes=False),
    scratch_types=[
        pltpu.VMEM((T * D,), jnp.float32),
        pltpu.VMEM((T * D,), jnp.float32),
        pltpu.VMEM((T + LANES,), jnp.int32),
        pltpu.VMEM((T + LANES,), jnp.int32),
        pltpu.VMEM((SEG_PER_W * D,), jnp.float32),
        pltpu.VMEM((SEG_PER_W * LANES,), jnp.float32),
        pltpu.VMEM((LANES,), jnp.float32),
        pltpu.VMEM((NW + LANES,), jnp.int32),
        pltpu.SMEM((8,), jnp.float32),
        pltpu.SemaphoreType.DMA,
        pltpu.SemaphoreType.DMA,
    ],
)


def kernel(h, pos, segment_ids, pos_weight):
    tidx = jnp.minimum(jnp.arange(LANES), pos_weight.shape[0] - 1)
    table = jax.nn.softplus(pos_weight[tidx, 0].astype(jnp.float32))
    packed = (segment_ids << 9) | pos
    bounds = jnp.arange(NW + 1, dtype=jnp.int32) * SEG_PER_W
    offs = jnp.sum(segment_ids[None, :] < bounds[:, None],
                   axis=1, dtype=jnp.int32)
    offs = jnp.pad(offs, (0, LANES - 1))
    out = _wmr(h.reshape(-1), packed, table, offs)
    return out.reshape(G, D)


# speedup vs baseline: 1.0271x; 1.0271x over previous
import jax
import jax.numpy as jnp
from jax import lax
from jax.experimental import pallas as pl
from jax.experimental.pallas import tpu as pltpu
from jax.experimental.pallas import tpu_sc as plsc

N = 320000
D = 128
G = 2048
NC = 2
NS = 16
NW = NC * NS
SEG_PER_W = G // NW
T = 400
LANES = 16
NVR = D // LANES


def _wmr_body(h_hbm, pk_hbm, table_hbm, offs_hbm, out_hbm,
              ht0, ht1, pk0, pk1, acc, dacc, table_v, offs_v,
              table_s,
              sem0, sem1):
    wid = lax.axis_index("s") * NC + lax.axis_index("c")
    g0d = wid * SEG_PER_W * D

    pltpu.sync_copy(table_hbm, table_v)
    pltpu.sync_copy(offs_hbm, offs_v)
    tv = table_v[pl.ds(0, LANES)]
    table_s[0] = tv[0]
    table_s[1] = tv[1]
    table_s[2] = tv[2]
    ov = offs_v[pl.ds(wid, LANES)]
    r0 = ov[0]
    r1 = ov[1]

    zeros = jnp.zeros((LANES,), jnp.float32)
    lane0_f = jnp.where(lax.iota(jnp.int32, LANES) == 0, 1.0, 0.0)

    t0 = r0 // T
    t1 = (r1 + T - 1) // T
    nt = t1 - t0

    bufs = ((ht0, pk0, sem0), (ht1, pk1, sem1))

    def issue(t, buf):
        htb, pkb, sem = buf
        base = t * T
        pltpu.async_copy(h_hbm.at[pl.ds(base * D, T * D)], htb, sem)
        pltpu.async_copy(pk_hbm.at[pl.ds(base, T)], pkb.at[pl.ds(0, T)], sem)

    def drain(buf):
        htb, pkb, sem = buf
        pltpu.make_async_copy(h_hbm.at[pl.ds(0, T * D)], htb, sem).wait()
        pltpu.make_async_copy(pk_hbm.at[pl.ds(0, T)], pkb.at[pl.ds(0, T)],
                              sem).wait()

    @plsc.parallel_loop(0, SEG_PER_W * D // LANES, unroll=8)
    def _(k):
        acc[pl.ds(k * LANES, LANES)] = zeros

    @plsc.parallel_loop(0, SEG_PER_W, unroll=8)
    def _(l):
        dacc[pl.ds(l * LANES, LANES)] = zeros

    @pl.when(nt > 0)
    def _():
        issue(t0, bufs[0])

    def do_row(htb, pk, i):
        p = pk & 3
        ao = (pk >> 2) - g0d
        a = table_s[p]
        plsc.addupdate(dacc.at[pl.ds(ao >> 3, LANES)], a * lane0_f)
        ho = i * D
        for j in range(NVR):
            plsc.addupdate(acc.at[pl.ds(ao + j * LANES, LANES)],
                           a * htb[pl.ds(ho + j * LANES, LANES)])

    def process(tt, buf):
        htb, pkb, _ = buf
        base = (t0 + tt) * T
        i_lo = jnp.maximum(r0 - base, 0)
        i_hi = jnp.minimum(r1 - base, T)
        a_lo = (i_lo + LANES - 1) & ~(LANES - 1)
        a_hi = i_hi & ~(LANES - 1)
        mid_end = jnp.minimum(a_lo, i_hi)
        tail_lo = jnp.maximum(a_hi, mid_end)
        blk_hi = jnp.maximum(a_lo, a_hi) >> 4

        @plsc.parallel_loop(i_lo, mid_end)
        def _(i):
            do_row(htb, pkb[pl.ds(i, LANES)][0], i)

        @plsc.parallel_loop(a_lo >> 4, blk_hi)
        def _(b):
            ib = b * LANES
            pkv = pkb[pl.ds(ib, LANES)]
            e0 = pkv[0]
            e15 = pkv[15]
            same = (e0 >> 9) == (e15 >> 9)

            @pl.when(same)
            def _():
                ao = (e0 >> 2) - g0d
                accs = [zeros] * NVR
                for r in range(LANES):
                    a = table_s[pkv[r] & 3]
                    ho = (ib + r) * D
                    for j in range(NVR):
                        accs[j] = accs[j] + a * htb[pl.ds(ho + j * LANES,
                                                          LANES)]
                for j in range(NVR):
                    plsc.addupdate(acc.at[pl.ds(ao + j * LANES, LANES)],
                                   accs[j])
                a16 = plsc.load_gather(table_v, [pkv & 3])
                plsc.addupdate(dacc.at[pl.ds(ao >> 3, LANES)], a16)

            @pl.when(jnp.logical_not(same))
            def _():
                seg0 = e0 >> 9
                seg15 = e15 >> 9
                aoA = (e0 >> 2) - g0d
                aoB = (e15 >> 2) - g0d
                accA = [zeros] * NVR
                accB = [zeros] * NVR
                bad = jnp.int32(0)
                for r in range(LANES):
                    pk = pkv[r]
                    seg_r = pk >> 9
                    a = table_s[pk & 3]
                    inA = seg_r == seg0
                    inB = seg_r == seg15
                    aA = jnp.where(inA, a, 0.0)
                    aB = jnp.where(inB, a, 0.0)
                    bad = bad | jnp.where(jnp.logical_or(inA, inB), 0, 1)
                    ho = (ib + r) * D
                    for j in range(NVR):
                        hv = htb[pl.ds(ho + j * LANES, LANES)]
                        accA[j] = accA[j] + aA * hv
                        accB[j] = accB[j] + aB * hv
                for j in range(NVR):
                    plsc.addupdate(acc.at[pl.ds(aoA + j * LANES, LANES)],
                                   accA[j])
                    plsc.addupdate(acc.at[pl.ds(aoB + j * LANES, LANES)],
                                   accB[j])
                svv = pkv >> 9
                a16 = plsc.load_gather(table_v, [pkv & 3])
                mA = svv == jnp.full((LANES,), seg0)
                mB = svv == jnp.full((LANES,), seg15)
                plsc.addupdate(dacc.at[pl.ds(aoA >> 3, LANES)],
                               jnp.where(mA, a16, 0.0))
                plsc.addupdate(dacc.at[pl.ds(aoB >> 3, LANES)],
                               jnp.where(mB, a16, 0.0))

                @pl.when(bad != 0)
                def _():
                    def mid_row(r, carry):
                        pk = pkb[pl.ds(ib + r, LANES)][0]
                        seg_r = pk >> 9

                        @pl.when(jnp.logical_and(seg_r != seg0,
                                                 seg_r != seg15))
                        def _():
                            do_row(htb, pk, ib + r)
                        return carry

                    lax.fori_loop(0, LANES, mid_row, 0)

        @plsc.parallel_loop(tail_lo, i_hi)
        def _(i):
            do_row(htb, pkb[pl.ds(i, LANES)][0], i)

    def tile_body(tt, carry):
        for k in (0, 1):
            @pl.when((tt & 1) == k)
            def _():
                drain(bufs[k])

                @pl.when(tt + 1 < nt)
                def _():
                    issue(t0 + tt + 1, bufs[1 - k])

                process(tt, bufs[k])
        return carry

    lax.fori_loop(0, nt, tile_body, 0)

    @plsc.parallel_loop(0, SEG_PER_W, unroll=2)
    def _(l):
        d = jnp.sum(dacc[pl.ds(l * LANES, LANES)])
        r16 = 1.0 / jnp.maximum(jnp.full((LANES,), d), 1e-12)
        for j in range(NVR):
            o = l * D + j * LANES
            acc[pl.ds(o, LANES)] = acc[pl.ds(o, LANES)] * r16

    pltpu.sync_copy(acc, out_hbm.at[pl.ds(wid * SEG_PER_W * D, SEG_PER_W * D)])


_wmr = pl.kernel(
    _wmr_body,
    mesh=plsc.VectorSubcoreMesh(core_axis_name="c", subcore_axis_name="s"),
    out_type=jax.ShapeDtypeStruct((G * D,), jnp.float32),
    compiler_params=pltpu.CompilerParams(needs_layout_passes=False),
    scratch_types=[
        pltpu.VMEM((T * D,), jnp.float32),
        pltpu.VMEM((T * D,), jnp.float32),
        pltpu.VMEM((T + LANES,), jnp.int32),
        pltpu.VMEM((T + LANES,), jnp.int32),
        pltpu.VMEM((SEG_PER_W * D,), jnp.float32),
        pltpu.VMEM((SEG_PER_W * LANES,), jnp.float32),
        pltpu.VMEM((LANES,), jnp.float32),
        pltpu.VMEM((NW + LANES,), jnp.int32),
        pltpu.SMEM((8,), jnp.float32),
        pltpu.SemaphoreType.DMA,
        pltpu.SemaphoreType.DMA,
    ],
)


def kernel(h, pos, segment_ids, pos_weight):
    tidx = jnp.minimum(jnp.arange(LANES), pos_weight.shape[0] - 1)
    table = jax.nn.softplus(pos_weight[tidx, 0].astype(jnp.float32))
    packed = (segment_ids << 9) | pos
    bounds = jnp.arange(NW + 1, dtype=jnp.int32) * SEG_PER_W
    offs = jnp.sum(segment_ids[None, :] < bounds[:, None],
                   axis=1, dtype=jnp.int32)
    offs = jnp.pad(offs, (0, LANES - 1))
    out = _wmr(h.reshape(-1), packed, table, offs)
    return out.reshape(G, D)
